# baseline (speedup 1.0000x reference)
import jax
import jax.numpy as jnp
from jax.experimental import pallas as pl
from jax.experimental.pallas import tpu as pltpu

NUM_TOKENS = 16384
D_MODEL = 2048
NUM_EXPERTS = 16
TOP_K = 2

BT = 1024


def _gate_block(x_ref, wt_ref, w_out_ref, idx_out_ref):
    logits = jnp.dot(x_ref[...], wt_ref[...], preferred_element_type=jnp.float32)
    m = jnp.max(logits, axis=1, keepdims=True)
    e = jnp.exp(logits - m)
    p = e / jnp.sum(e, axis=1, keepdims=True)

    iota = jax.lax.broadcasted_iota(jnp.int32, p.shape, 1)
    p1 = jnp.max(p, axis=1, keepdims=True)
    i1 = jnp.min(jnp.where(p == p1, iota, NUM_EXPERTS), axis=1, keepdims=True)
    masked = jnp.where(iota == i1, -1.0, p)
    p2 = jnp.max(masked, axis=1, keepdims=True)
    i2 = jnp.min(jnp.where(masked == p2, iota, NUM_EXPERTS), axis=1, keepdims=True)

    s = p1 + p2
    w_out_ref[:, 0:1] = p1 / s
    w_out_ref[:, 1:2] = p2 / s
    idx_out_ref[:, 0:1] = i1
    idx_out_ref[:, 1:2] = i2


def kernel(x, W):
    wt = W.T
    grid = (NUM_TOKENS // BT,)
    w_out, idx_out = pl.pallas_call(
        _gate_block,
        grid=grid,
        in_specs=[
            pl.BlockSpec((BT, D_MODEL), lambda i: (i, 0)),
            pl.BlockSpec((D_MODEL, NUM_EXPERTS), lambda i: (0, 0)),
        ],
        out_specs=[
            pl.BlockSpec((BT, TOP_K), lambda i: (i, 0)),
            pl.BlockSpec((BT, TOP_K), lambda i: (i, 0)),
        ],
        out_shape=[
            jax.ShapeDtypeStruct((NUM_TOKENS, TOP_K), jnp.float32),
            jax.ShapeDtypeStruct((NUM_TOKENS, TOP_K), jnp.int32),
        ],
        compiler_params=pltpu.CompilerParams(
            dimension_semantics=("arbitrary",),
        ),
    )(x, wt)
    return (w_out, idx_out)
